# Initial kernel scaffold; baseline (speedup 1.0000x reference)
import jax
import jax.numpy as jnp
from jax.experimental import pallas as pl


def kernel(x, edge_index, W1, b1, W2, b2):
    raise NotImplementedError("write your pallas kernel here")


# speedup vs baseline: 5.0281x; 5.0281x over previous
import functools

import jax
import jax.numpy as jnp
from jax import lax
from jax.experimental import pallas as pl
from jax.experimental.pallas import tpu as pltpu
from jax.experimental.pallas import tpu_sc as plsc

N = 10000
D = 128
E = 320000
NC = 2
NS = 16
NW = NC * NS
EPW = E // NW
C = 80
NCHUNK = EPW // C
RPT = 624
TAIL = N - NS * RPT
SUB = 104
NSUB = RPT // SUB
DEGW = 16


def _make_sc_agg(with_deg):
    out_type = [jax.ShapeDtypeStruct((NC * N, D), jnp.float32)]
    scratch = [
        pltpu.VMEM_SHARED((N, D), jnp.float32),
        pltpu.VMEM((C,), jnp.int32),
        pltpu.VMEM((C,), jnp.int32),
        pltpu.VMEM((C, D), jnp.float32),
        pltpu.VMEM((SUB, D), jnp.float32),
        pltpu.SemaphoreType.DMA,
    ]
    if with_deg:
        out_type.append(jax.ShapeDtypeStruct((NC * N, DEGW), jnp.float32))
        scratch += [
            pltpu.VMEM_SHARED((N, DEGW), jnp.float32),
            pltpu.VMEM((C, DEGW), jnp.float32),
            pltpu.VMEM((SUB, DEGW), jnp.float32),
        ]

    mesh = plsc.VectorSubcoreMesh(core_axis_name="c", subcore_axis_name="s")

    def body(x_hbm, src_hbm, dst_hbm, zrow_hbm, zdeg_hbm, ones_hbm,
             agg_out, deg_out, agg_sh, srcv, dstv, rows, zbuf, sem,
             deg_sh=None, onesv=None, zdbuf=None):
        cid = lax.axis_index("c")
        sid = lax.axis_index("s")
        wid = sid * NC + cid

        pltpu.sync_copy(zrow_hbm, zbuf)
        if with_deg:
            pltpu.sync_copy(zdeg_hbm, zdbuf)
            pltpu.sync_copy(ones_hbm, onesv)

        @pl.loop(0, NSUB)
        def _(j):
            r0 = sid * RPT + j * SUB
            pltpu.sync_copy(zbuf, agg_sh.at[pl.ds(r0, SUB)])
            if with_deg:
                pltpu.sync_copy(zdbuf, deg_sh.at[pl.ds(r0, SUB)])

        @pl.when(sid == 0)
        def _():
            pltpu.sync_copy(zbuf.at[pl.ds(0, TAIL)],
                            agg_sh.at[pl.ds(NS * RPT, TAIL)])
            if with_deg:
                pltpu.sync_copy(zdbuf.at[pl.ds(0, TAIL)],
                                deg_sh.at[pl.ds(NS * RPT, TAIL)])

        plsc.subcore_barrier()

        base = wid * EPW

        @pl.loop(0, NCHUNK)
        def _(i):
            off = base + i * C
            pltpu.sync_copy(src_hbm.at[pl.ds(off, C)], srcv)
            pltpu.sync_copy(dst_hbm.at[pl.ds(off, C)], dstv)
            pltpu.async_copy(x_hbm.at[srcv], rows, sem).wait()
            pltpu.sync_copy(rows, agg_sh.at[dstv], add=True)
            if with_deg:
                pltpu.sync_copy(onesv, deg_sh.at[dstv], add=True)

        plsc.subcore_barrier()

        @pl.loop(0, NSUB)
        def _(j):
            r0 = sid * RPT + j * SUB
            pltpu.sync_copy(agg_sh.at[pl.ds(r0, SUB)], zbuf)
            pltpu.sync_copy(zbuf, agg_out.at[pl.ds(cid * N + r0, SUB)])
            if with_deg:
                pltpu.sync_copy(deg_sh.at[pl.ds(r0, SUB)], zdbuf)
                pltpu.sync_copy(zdbuf, deg_out.at[pl.ds(cid * N + r0, SUB)])

        @pl.when(sid == 0)
        def _():
            t0 = NS * RPT
            pltpu.sync_copy(agg_sh.at[pl.ds(t0, TAIL)],
                            zbuf.at[pl.ds(0, TAIL)])
            pltpu.sync_copy(zbuf.at[pl.ds(0, TAIL)],
                            agg_out.at[pl.ds(cid * N + t0, TAIL)])
            if with_deg:
                pltpu.sync_copy(deg_sh.at[pl.ds(t0, TAIL)],
                                zdbuf.at[pl.ds(0, TAIL)])
                pltpu.sync_copy(zdbuf.at[pl.ds(0, TAIL)],
                                deg_out.at[pl.ds(cid * N + t0, TAIL)])

    if with_deg:
        def body_w(x_hbm, src_hbm, dst_hbm, zrow_hbm, zdeg_hbm, ones_hbm,
                   agg_out, deg_out, agg_sh, srcv, dstv, rows, zbuf, sem,
                   deg_sh, onesv, zdbuf):
            body(x_hbm, src_hbm, dst_hbm, zrow_hbm, zdeg_hbm, ones_hbm,
                 agg_out, deg_out, agg_sh, srcv, dstv, rows, zbuf, sem,
                 deg_sh, onesv, zdbuf)
    else:
        def body_w(x_hbm, src_hbm, dst_hbm, zrow_hbm, zdeg_hbm, ones_hbm,
                   agg_out, agg_sh, srcv, dstv, rows, zbuf, sem):
            body(x_hbm, src_hbm, dst_hbm, zrow_hbm, zdeg_hbm, ones_hbm,
                 agg_out, None, agg_sh, srcv, dstv, rows, zbuf, sem)

    return pl.kernel(body_w, out_type=tuple(out_type), mesh=mesh,
                     scratch_types=scratch,
                     compiler_params=pltpu.CompilerParams(
                         use_tc_tiling_on_sc=False))


_sc_agg_deg = _make_sc_agg(True)
_sc_agg = _make_sc_agg(False)


_TC_R = 2000


def _tc_body(agg_ref, deg_ref, x_ref, w_ref, b_ref, o_ref):
    a = agg_ref[0] + agg_ref[1]
    dg = deg_ref[0, :, 0:1] + deg_ref[1, :, 0:1]
    a = a / jnp.maximum(dg, 1.0)
    h = jnp.dot(a, w_ref[...], preferred_element_type=jnp.float32)
    o_ref[...] = jnp.maximum(h + b_ref[...], 0.0) + x_ref[...]


def _tc_layer(agg, deg, x, W, b):
    return pl.pallas_call(
        _tc_body,
        out_shape=jax.ShapeDtypeStruct((N, D), jnp.float32),
        grid=(N // _TC_R,),
        in_specs=[
            pl.BlockSpec((NC, _TC_R, D), lambda i: (0, i, 0)),
            pl.BlockSpec((NC, _TC_R, DEGW), lambda i: (0, i, 0)),
            pl.BlockSpec((_TC_R, D), lambda i: (i, 0)),
            pl.BlockSpec((D, D), lambda i: (0, 0)),
            pl.BlockSpec((1, D), lambda i: (0, 0)),
        ],
        out_specs=pl.BlockSpec((_TC_R, D), lambda i: (i, 0)),
    )(agg, deg, x, W, b)


def kernel(x, edge_index, W1, b1, W2, b2):
    src = edge_index[0].astype(jnp.int32)
    dst = edge_index[1].astype(jnp.int32)
    zrow = jnp.zeros((SUB, D), jnp.float32)
    zdeg = jnp.zeros((SUB, DEGW), jnp.float32)
    ones = jnp.ones((C, DEGW), jnp.float32)
    b1r = b1.reshape(1, D)
    b2r = b2.reshape(1, D)

    agg1, deg1 = _sc_agg_deg(x, src, dst, zrow, zdeg, ones)
    agg1 = agg1.reshape(NC, N, D)
    deg = deg1.reshape(NC, N, DEGW)
    h1 = _tc_layer(agg1, deg, x, W1, b1r)

    (agg2,) = _sc_agg(h1, src, dst, zrow, zdeg, ones)
    agg2 = agg2.reshape(NC, N, D)
    out = _tc_layer(agg2, deg, h1, W2, b2r)
    return out


# speedup vs baseline: 8.1602x; 1.6229x over previous
import jax
import jax.numpy as jnp
from jax import lax
from jax.experimental import pallas as pl
from jax.experimental.pallas import tpu as pltpu
from jax.experimental.pallas import tpu_sc as plsc

N = 10000
D = 128
E = 320000
NC = 2
NS = 16
NW = NC * NS
EPW = E // NW
C = 80
NCHUNK = EPW // C
RPT = 624
TAIL = N - NS * RPT
SUB = 48
NSUB = RPT // SUB
CV = C // 16
CREM = C - CV * 16


def _make_sc_agg(with_deg):
    out_type = [jax.ShapeDtypeStruct((NC * N, D), jnp.float32)]
    scratch = [
        pltpu.VMEM_SHARED((N, D), jnp.float32),
        pltpu.VMEM((C,), jnp.int32),
        pltpu.VMEM((C,), jnp.int32),
        pltpu.VMEM((C,), jnp.int32),
        pltpu.VMEM((C,), jnp.int32),
        pltpu.VMEM((C, D), jnp.float32),
        pltpu.VMEM((C, D), jnp.float32),
        pltpu.VMEM((SUB, D), jnp.float32),
        pltpu.SemaphoreType.DMA,
        pltpu.SemaphoreType.DMA,
    ]
    if with_deg:
        out_type.append(jax.ShapeDtypeStruct((NW, N), jnp.float32))
        scratch.append(pltpu.VMEM((N,), jnp.float32))

    mesh = plsc.VectorSubcoreMesh(core_axis_name="c", subcore_axis_name="s")

    def body(x_hbm, src_hbm, dst_hbm, zrow_hbm,
             agg_out, deg_out, agg_sh, sbuf0, sbuf1, dbuf0, dbuf1,
             rows0, rows1, zbuf, sem0, sem1, degloc=None):
        cid = lax.axis_index("c")
        sid = lax.axis_index("s")
        wid = sid * NC + cid

        pltpu.sync_copy(zrow_hbm, zbuf)

        @pl.loop(0, NSUB)
        def _(j):
            pltpu.sync_copy(zbuf, agg_sh.at[pl.ds(sid * RPT + j * SUB, SUB)])

        @pl.when(sid == 0)
        def _():
            pltpu.sync_copy(zbuf.at[pl.ds(0, TAIL)],
                            agg_sh.at[pl.ds(NS * RPT, TAIL)])

        if with_deg:
            zv = jnp.zeros((16,), jnp.float32)

            @pl.loop(0, N // 16)
            def _(k):
                degloc[pl.ds(k * 16, 16)] = zv

        plsc.subcore_barrier()

        ones16 = jnp.ones((16,), jnp.float32)

        def deg_update(dbuf):
            for g in range(CV):
                idx = dbuf[pl.ds(g * 16, 16)]
                plsc.addupdate_scatter(degloc, [idx], ones16)

        pltpu.sync_copy(src_hbm.at[wid].at[0], sbuf0)
        pltpu.sync_copy(dst_hbm.at[wid].at[0], dbuf0)
        g0 = pltpu.async_copy(x_hbm.at[sbuf0], rows0, sem0)
        pltpu.sync_copy(src_hbm.at[wid].at[1], sbuf1)
        pltpu.sync_copy(dst_hbm.at[wid].at[1], dbuf1)
        g1 = pltpu.async_copy(x_hbm.at[sbuf1], rows1, sem1)

        @pl.loop(0, NCHUNK - 1, step=2)
        def _(i):
            g0.wait()
            pltpu.sync_copy(rows0, agg_sh.at[dbuf0], add=True)
            if with_deg:
                deg_update(dbuf0)

            @pl.when(i + 2 < NCHUNK)
            def _():
                pltpu.sync_copy(src_hbm.at[wid].at[i + 2], sbuf0)
                pltpu.sync_copy(dst_hbm.at[wid].at[i + 2], dbuf0)
                pltpu.async_copy(x_hbm.at[sbuf0], rows0, sem0)

            g1.wait()
            pltpu.sync_copy(rows1, agg_sh.at[dbuf1], add=True)
            if with_deg:
                deg_update(dbuf1)

            @pl.when(i + 3 < NCHUNK)
            def _():
                pltpu.sync_copy(src_hbm.at[wid].at[i + 3], sbuf1)
                pltpu.sync_copy(dst_hbm.at[wid].at[i + 3], dbuf1)
                pltpu.async_copy(x_hbm.at[sbuf1], rows1, sem1)

        g0.wait()
        pltpu.sync_copy(rows0, agg_sh.at[dbuf0], add=True)
        if with_deg:
            deg_update(dbuf0)

        plsc.subcore_barrier()

        @pl.loop(0, NSUB)
        def _(j):
            r0 = sid * RPT + j * SUB
            pltpu.sync_copy(agg_sh.at[pl.ds(r0, SUB)], zbuf)
            pltpu.sync_copy(zbuf, agg_out.at[pl.ds(cid * N + r0, SUB)])

        @pl.when(sid == 0)
        def _():
            t0 = NS * RPT
            pltpu.sync_copy(agg_sh.at[pl.ds(t0, TAIL)],
                            zbuf.at[pl.ds(0, TAIL)])
            pltpu.sync_copy(zbuf.at[pl.ds(0, TAIL)],
                            agg_out.at[pl.ds(cid * N + t0, TAIL)])

        if with_deg:
            pltpu.sync_copy(degloc, deg_out.at[wid])

    if with_deg:
        def body_w(x_hbm, src_hbm, dst_hbm, zrow_hbm, agg_out, deg_out,
                   agg_sh, sbuf0, sbuf1, dbuf0, dbuf1, rows0, rows1,
                   zbuf, sem0, sem1, degloc):
            body(x_hbm, src_hbm, dst_hbm, zrow_hbm, agg_out, deg_out,
                 agg_sh, sbuf0, sbuf1, dbuf0, dbuf1, rows0, rows1,
                 zbuf, sem0, sem1, degloc)
    else:
        def body_w(x_hbm, src_hbm, dst_hbm, zrow_hbm, agg_out,
                   agg_sh, sbuf0, sbuf1, dbuf0, dbuf1, rows0, rows1,
                   zbuf, sem0, sem1):
            body(x_hbm, src_hbm, dst_hbm, zrow_hbm, agg_out, None,
                 agg_sh, sbuf0, sbuf1, dbuf0, dbuf1, rows0, rows1,
                 zbuf, sem0, sem1)

    return pl.kernel(body_w, out_type=tuple(out_type), mesh=mesh,
                     scratch_types=scratch,
                     compiler_params=pltpu.CompilerParams(
                         use_tc_tiling_on_sc=False,
                         needs_layout_passes=False))


_sc_agg_deg = _make_sc_agg(True)
_sc_agg = _make_sc_agg(False)


_TC_R = 2000


def _tc_body(agg_ref, deg_ref, x_ref, w_ref, b_ref, o_ref):
    a = agg_ref[0] + agg_ref[1]
    dg = jnp.sum(deg_ref[...], axis=1, keepdims=True)
    a = a / jnp.maximum(dg, 1.0)
    h = jnp.dot(a, w_ref[...], preferred_element_type=jnp.float32)
    o_ref[...] = jnp.maximum(h + b_ref[...], 0.0) + x_ref[...]


def _tc_layer(agg, degT, x, W, b):
    return pl.pallas_call(
        _tc_body,
        out_shape=jax.ShapeDtypeStruct((N, D), jnp.float32),
        grid=(N // _TC_R,),
        in_specs=[
            pl.BlockSpec((NC, _TC_R, D), lambda i: (0, i, 0)),
            pl.BlockSpec((_TC_R, NW), lambda i: (i, 0)),
            pl.BlockSpec((_TC_R, D), lambda i: (i, 0)),
            pl.BlockSpec((D, D), lambda i: (0, 0)),
            pl.BlockSpec((1, D), lambda i: (0, 0)),
        ],
        out_specs=pl.BlockSpec((_TC_R, D), lambda i: (i, 0)),
    )(agg, degT, x, W, b)


def kernel(x, edge_index, W1, b1, W2, b2):
    src = edge_index[0].astype(jnp.int32).reshape(NW, NCHUNK, C)
    dst = edge_index[1].astype(jnp.int32).reshape(NW, NCHUNK, C)
    zrow = jnp.zeros((SUB, D), jnp.float32)
    b1r = b1.reshape(1, D)
    b2r = b2.reshape(1, D)

    agg1, degp = _sc_agg_deg(x, src, dst, zrow)
    agg1 = agg1.reshape(NC, N, D)
    degT = degp.T
    h1 = _tc_layer(agg1, degT, x, W1, b1r)

    (agg2,) = _sc_agg(h1, src, dst, zrow)
    agg2 = agg2.reshape(NC, N, D)
    out = _tc_layer(agg2, degT, h1, W2, b2r)
    return out


# speedup vs baseline: 9.4985x; 1.1640x over previous
import jax
import jax.numpy as jnp
from jax import lax
from jax.experimental import pallas as pl
from jax.experimental.pallas import tpu as pltpu
from jax.experimental.pallas import tpu_sc as plsc

N = 10000
D = 128
E = 320000
NC = 2
NS = 16
NW = NC * NS
EPW = E // NW
C = 80
NCHUNK = EPW // C
RPT = 624
TAIL = N - NS * RPT
SUB = 48
NSUB = RPT // SUB
CV = C // 16
CREM = C - CV * 16


def _make_sc_agg(with_deg):
    out_type = [jax.ShapeDtypeStruct((NC * N, D), jnp.float32)]
    scratch = [
        pltpu.VMEM_SHARED((N, D), jnp.float32),
        pltpu.VMEM((2, C), jnp.int32),
        pltpu.VMEM((2, C), jnp.int32),
        pltpu.VMEM((C, D), jnp.float32),
        pltpu.VMEM((C, D), jnp.float32),
        pltpu.VMEM((SUB, D), jnp.float32),
        pltpu.SemaphoreType.DMA,
        pltpu.SemaphoreType.DMA,
        pltpu.SemaphoreType.DMA,
        pltpu.SemaphoreType.DMA,
    ]
    if with_deg:
        out_type.append(jax.ShapeDtypeStruct((NW, N), jnp.float32))
        scratch.append(pltpu.VMEM((N,), jnp.float32))

    mesh = plsc.VectorSubcoreMesh(core_axis_name="c", subcore_axis_name="s")

    def body(x_hbm, sd_hbm, zrow_hbm,
             agg_out, deg_out, agg_sh, sd0, sd1,
             rows0, rows1, zbuf, gsem0, gsem1, isem0, isem1,
             degloc=None):
        cid = lax.axis_index("c")
        sid = lax.axis_index("s")
        wid = sid * NC + cid

        pltpu.sync_copy(sd_hbm.at[wid].at[0], sd0)
        g0 = pltpu.async_copy(x_hbm.at[sd0.at[0]], rows0, gsem0)
        pltpu.sync_copy(sd_hbm.at[wid].at[1], sd1)
        g1 = pltpu.async_copy(x_hbm.at[sd1.at[0]], rows1, gsem1)

        pltpu.sync_copy(zrow_hbm, zbuf)

        @pl.loop(0, NSUB)
        def _(j):
            pltpu.sync_copy(zbuf, agg_sh.at[pl.ds(sid * RPT + j * SUB, SUB)])

        @pl.when(sid == 0)
        def _():
            pltpu.sync_copy(zbuf.at[pl.ds(0, TAIL)],
                            agg_sh.at[pl.ds(NS * RPT, TAIL)])

        if with_deg:
            zv = jnp.zeros((16,), jnp.float32)

            @pl.loop(0, N // 16)
            def _(k):
                degloc[pl.ds(k * 16, 16)] = zv

        plsc.subcore_barrier()

        ones16 = jnp.ones((16,), jnp.float32)

        def deg_update(sd):
            for g in range(CV):
                idx = sd[1, pl.ds(g * 16, 16)]
                plsc.addupdate_scatter(degloc, [idx], ones16)

        @pl.loop(0, NCHUNK - 1, step=2)
        def _(i):
            g0.wait()
            pltpu.sync_copy(rows0, agg_sh.at[sd0.at[1]], add=True)
            if with_deg:
                deg_update(sd0)

            @pl.when(i + 2 < NCHUNK)
            def _():
                pltpu.async_copy(sd_hbm.at[wid].at[i + 2], sd0, isem0)

            g1.wait()
            pltpu.sync_copy(rows1, agg_sh.at[sd1.at[1]], add=True)
            if with_deg:
                deg_update(sd1)

            @pl.when(i + 2 < NCHUNK)
            def _():
                pltpu.make_async_copy(sd_hbm.at[wid].at[i + 2], sd0,
                                      isem0).wait()
                pltpu.async_copy(x_hbm.at[sd0.at[0]], rows0, gsem0)

            @pl.when(i + 3 < NCHUNK)
            def _():
                pltpu.async_copy(sd_hbm.at[wid].at[i + 3], sd1, isem1)
                pltpu.make_async_copy(sd_hbm.at[wid].at[i + 3], sd1,
                                      isem1).wait()
                pltpu.async_copy(x_hbm.at[sd1.at[0]], rows1, gsem1)

        g0.wait()
        pltpu.sync_copy(rows0, agg_sh.at[sd0.at[1]], add=True)
        if with_deg:
            deg_update(sd0)

        plsc.subcore_barrier()

        @pl.loop(0, NSUB)
        def _(j):
            r0 = sid * RPT + j * SUB
            pltpu.sync_copy(agg_sh.at[pl.ds(r0, SUB)], zbuf)
            pltpu.sync_copy(zbuf, agg_out.at[pl.ds(cid * N + r0, SUB)])

        @pl.when(sid == 0)
        def _():
            t0 = NS * RPT
            pltpu.sync_copy(agg_sh.at[pl.ds(t0, TAIL)],
                            zbuf.at[pl.ds(0, TAIL)])
            pltpu.sync_copy(zbuf.at[pl.ds(0, TAIL)],
                            agg_out.at[pl.ds(cid * N + t0, TAIL)])

        if with_deg:
            pltpu.sync_copy(degloc, deg_out.at[wid])

    if with_deg:
        def body_w(x_hbm, sd_hbm, zrow_hbm, agg_out, deg_out,
                   agg_sh, sd0, sd1, rows0, rows1, zbuf,
                   gsem0, gsem1, isem0, isem1, degloc):
            body(x_hbm, sd_hbm, zrow_hbm, agg_out, deg_out,
                 agg_sh, sd0, sd1, rows0, rows1, zbuf,
                 gsem0, gsem1, isem0, isem1, degloc)
    else:
        def body_w(x_hbm, sd_hbm, zrow_hbm, agg_out,
                   agg_sh, sd0, sd1, rows0, rows1, zbuf,
                   gsem0, gsem1, isem0, isem1):
            body(x_hbm, sd_hbm, zrow_hbm, agg_out, None,
                 agg_sh, sd0, sd1, rows0, rows1, zbuf,
                 gsem0, gsem1, isem0, isem1)

    return pl.kernel(body_w, out_type=tuple(out_type), mesh=mesh,
                     scratch_types=scratch,
                     compiler_params=pltpu.CompilerParams(
                         use_tc_tiling_on_sc=False,
                         needs_layout_passes=False))


_sc_agg_deg = _make_sc_agg(True)
_sc_agg = _make_sc_agg(False)


_TC_R = 2000


def _tc_body(agg_ref, deg_ref, x_ref, w_ref, b_ref, o_ref):
    a = agg_ref[0] + agg_ref[1]
    dg = jnp.sum(deg_ref[...], axis=1, keepdims=True)
    a = a / jnp.maximum(dg, 1.0)
    h = jnp.dot(a, w_ref[...], preferred_element_type=jnp.float32)
    o_ref[...] = jnp.maximum(h + b_ref[...], 0.0) + x_ref[...]


def _tc_layer(agg, degT, x, W, b):
    return pl.pallas_call(
        _tc_body,
        out_shape=jax.ShapeDtypeStruct((N, D), jnp.float32),
        grid=(N // _TC_R,),
        in_specs=[
            pl.BlockSpec((NC, _TC_R, D), lambda i: (0, i, 0)),
            pl.BlockSpec((_TC_R, NW), lambda i: (i, 0)),
            pl.BlockSpec((_TC_R, D), lambda i: (i, 0)),
            pl.BlockSpec((D, D), lambda i: (0, 0)),
            pl.BlockSpec((1, D), lambda i: (0, 0)),
        ],
        out_specs=pl.BlockSpec((_TC_R, D), lambda i: (i, 0)),
    )(agg, degT, x, W, b)


def kernel(x, edge_index, W1, b1, W2, b2):
    ei = edge_index.astype(jnp.int32).reshape(2, NW, NCHUNK, C)
    sd = ei.transpose(1, 2, 0, 3)
    zrow = jnp.zeros((SUB, D), jnp.float32)
    b1r = b1.reshape(1, D)
    b2r = b2.reshape(1, D)

    agg1, degp = _sc_agg_deg(x, sd, zrow)
    agg1 = agg1.reshape(NC, N, D)
    degT = degp.T
    h1 = _tc_layer(agg1, degT, x, W1, b1r)

    (agg2,) = _sc_agg(h1, sd, zrow)
    agg2 = agg2.reshape(NC, N, D)
    out = _tc_layer(agg2, degT, h1, W2, b2r)
    return out
